# baseline (speedup 1.0000x reference)
import functools

import jax
import jax.numpy as jnp
from jax import lax
from jax.experimental import pallas as pl
from jax.experimental.pallas import tpu as pltpu
from jax.experimental.pallas import tpu_sc as plsc

_VOCAB = 1000
_VPAD = 1024
_NEMB = 32
_T = 8
_BATCH = 4096
_TOK = _BATCH * _T

_NC = 2
_NS = 16
_L = 16
_NW = _NC * _NS
_RPW = _TOK // _NW
_CH = 64
_NCH = _RPW // _CH


def _table_body(tt_ref, pos_ref, w_ref, b_ref, out_ref):
    e = jnp.dot(tt_ref[...], w_ref[...], preferred_element_type=jnp.float32)
    p = jnp.dot(pos_ref[...], w_ref[...], preferred_element_type=jnp.float32)
    p = p + b_ref[...]
    out_ref[...] = e[:, None, :] + p[None, :, :]


def _build_table(token_table, pos_table, w, bias):
    vb = 200
    grid = _VOCAB // vb
    w_pad = jnp.pad(w, ((0, 0), (0, _VPAD - _VOCAB)))
    b_pad = jnp.pad(bias.reshape(1, _VOCAB), ((0, 0), (0, _VPAD - _VOCAB)))
    out = pl.pallas_call(
        _table_body,
        grid=(grid,),
        in_specs=[
            pl.BlockSpec((vb, _NEMB), lambda i: (i, 0)),
            pl.BlockSpec((_T, _NEMB), lambda i: (0, 0)),
            pl.BlockSpec((_NEMB, _VPAD), lambda i: (0, 0)),
            pl.BlockSpec((1, _VPAD), lambda i: (0, 0)),
        ],
        out_specs=pl.BlockSpec((vb, _T, _VPAD), lambda i: (i, 0, 0)),
        out_shape=jax.ShapeDtypeStruct((_VOCAB, _T, _VPAD), jnp.float32),
    )(token_table, pos_table, w_pad, b_pad)
    return out.reshape(_VOCAB * _T, _VPAD)


def _gather_body(gp_hbm, idx_hbm, out_hbm, idx_v, cidx_v, rows_v, sem):
    wid = lax.axis_index("s") * _NC + lax.axis_index("c")
    base = wid * _RPW
    pltpu.sync_copy(idx_hbm.at[pl.ds(base, _RPW)], idx_v)
    tpat = lax.rem(lax.iota(jnp.int32, _L), jnp.full((_L,), _T, jnp.int32))

    def cbody(i, carry):
        v = idx_v[pl.ds(i * _L, _L)]
        cidx_v[pl.ds(i * _L, _L)] = (v << 3) + tpat
        return carry

    lax.fori_loop(0, _RPW // _L, cbody, 0)

    for c in range(_NCH):
        pltpu.async_copy(
            gp_hbm.at[cidx_v.at[pl.ds(c * _CH, _CH)]], rows_v, sem
        ).wait()
        pltpu.sync_copy(
            rows_v.at[:, pl.ds(0, _VOCAB)],
            out_hbm.at[pl.ds(base + c * _CH, _CH)],
        )


def _gather(gp, idx_flat):
    mesh = plsc.VectorSubcoreMesh(
        core_axis_name="c", subcore_axis_name="s",
        num_cores=_NC, num_subcores=_NS,
    )
    run = functools.partial(
        pl.kernel,
        out_type=jax.ShapeDtypeStruct((_TOK, _VOCAB), jnp.float32),
        mesh=mesh,
        compiler_params=pltpu.CompilerParams(use_tc_tiling_on_sc=False),
        scratch_types=[
            pltpu.VMEM((_RPW,), jnp.int32),
            pltpu.VMEM((_RPW,), jnp.int32),
            pltpu.VMEM((_CH, _VPAD), jnp.float32),
            pltpu.SemaphoreType.DMA,
        ],
    )(_gather_body)
    return run(gp, idx_flat)


def kernel(idx, token_table, pos_table, W, b):
    gp = _build_table(token_table, pos_table, W, b)
    idx_flat = idx.reshape(_TOK).astype(jnp.int32)
    out = _gather(gp, idx_flat)
    return out.reshape(_BATCH, _T, _VOCAB)


# speedup vs baseline: 1.1967x; 1.1967x over previous
import functools

import jax
import jax.numpy as jnp
from jax import lax
from jax.experimental import pallas as pl
from jax.experimental.pallas import tpu as pltpu
from jax.experimental.pallas import tpu_sc as plsc

_VOCAB = 1000
_VPAD = 1024
_NEMB = 32
_T = 8
_BATCH = 4096
_TOK = _BATCH * _T

_NC = 2
_NS = 16
_L = 16
_NW = _NC * _NS
_RPW = _TOK // _NW
_CH = 16
_NCH = _RPW // _CH


def _table_body(tt_ref, pos_ref, w_ref, b_ref, out_ref):
    e = jnp.dot(tt_ref[...], w_ref[...], preferred_element_type=jnp.float32)
    p = jnp.dot(pos_ref[...], w_ref[...], preferred_element_type=jnp.float32)
    p = p + b_ref[...]
    out_ref[...] = e[:, None, :] + p[None, :, :]


def _build_table(token_table, pos_table, w, bias):
    vb = 200
    grid = _VOCAB // vb
    w_pad = jnp.pad(w, ((0, 0), (0, _VPAD - _VOCAB)))
    b_pad = jnp.pad(bias.reshape(1, _VOCAB), ((0, 0), (0, _VPAD - _VOCAB)))
    out = pl.pallas_call(
        _table_body,
        grid=(grid,),
        in_specs=[
            pl.BlockSpec((vb, _NEMB), lambda i: (i, 0)),
            pl.BlockSpec((_T, _NEMB), lambda i: (0, 0)),
            pl.BlockSpec((_NEMB, _VPAD), lambda i: (0, 0)),
            pl.BlockSpec((1, _VPAD), lambda i: (0, 0)),
        ],
        out_specs=pl.BlockSpec((vb, _T, _VPAD), lambda i: (i, 0, 0)),
        out_shape=jax.ShapeDtypeStruct((_VOCAB, _T, _VPAD), jnp.float32),
    )(token_table, pos_table, w_pad, b_pad)
    return out.reshape(_VOCAB * _T, _VPAD)


def _narrow_rows(rows_ref, out_ref):
    lanes = lax.iota(jnp.int32, _L)
    tail_mask = lanes < 8
    tail_cols = jnp.full((_L,), 992, jnp.int32) + lanes

    def row_body(j, carry):
        for k in range(_VOCAB // _L):
            out_ref[j, pl.ds(k * _L, _L)] = rows_ref[j, pl.ds(k * _L, _L)]
        tail = rows_ref[j, pl.ds(992, _L)]
        plsc.store_scatter(
            out_ref,
            [jnp.full((_L,), j, jnp.int32), tail_cols],
            tail,
            mask=tail_mask,
        )
        return carry

    lax.fori_loop(0, _CH, row_body, 0)


def _gather_body(gp_hbm, idx_hbm, out_hbm, idx_v, cidx_v, rows_v, out_v, gsem, osem):
    wid = lax.axis_index("s") * _NC + lax.axis_index("c")
    base = wid * _RPW
    pltpu.sync_copy(idx_hbm.at[pl.ds(base, _RPW)], idx_v)
    tpat = lax.rem(lax.iota(jnp.int32, _L), jnp.full((_L,), _T, jnp.int32))

    def cbody(i, carry):
        v = idx_v[pl.ds(i * _L, _L)]
        cidx_v[pl.ds(i * _L, _L)] = (v << 3) + tpat
        return carry

    lax.fori_loop(0, _RPW // _L, cbody, 0)

    def chunk_body(c, carry):
        start = pl.multiple_of(c * _CH, _CH)
        pltpu.async_copy(
            gp_hbm.at[cidx_v.at[pl.ds(start, _CH)]], rows_v, gsem
        ).wait()
        _narrow_rows(rows_v, out_v)
        pltpu.sync_copy(out_v, out_hbm.at[pl.ds(base + start, _CH)])
        return carry

    lax.fori_loop(0, _NCH, chunk_body, 0)


def _gather(gp, idx_flat):
    mesh = plsc.VectorSubcoreMesh(
        core_axis_name="c", subcore_axis_name="s",
        num_cores=_NC, num_subcores=_NS,
    )
    run = functools.partial(
        pl.kernel,
        out_type=jax.ShapeDtypeStruct((_TOK, _VOCAB), jnp.float32),
        mesh=mesh,
        compiler_params=pltpu.CompilerParams(needs_layout_passes=False),
        scratch_types=[
            pltpu.VMEM((_RPW,), jnp.int32),
            pltpu.VMEM((_RPW,), jnp.int32),
            pltpu.VMEM((_CH, _VPAD), jnp.float32),
            pltpu.VMEM((_CH, _VOCAB), jnp.float32),
            pltpu.SemaphoreType.DMA,
            pltpu.SemaphoreType.DMA,
        ],
    )(_gather_body)
    return run(gp, idx_flat)


def kernel(idx, token_table, pos_table, W, b):
    gp = _build_table(token_table, pos_table, W, b)
    idx_flat = idx.reshape(_TOK).astype(jnp.int32)
    out = _gather(gp, idx_flat)
    return out.reshape(_BATCH, _T, _VOCAB)
